# Initial kernel scaffold; baseline (speedup 1.0000x reference)
import jax
import jax.numpy as jnp
from jax.experimental import pallas as pl


def kernel(edge_index, rel_type, norm, W0, W1, W2):
    raise NotImplementedError("write your pallas kernel here")


# speedup vs baseline: 1.8116x; 1.8116x over previous
import functools

import jax
import jax.numpy as jnp
from jax.experimental import pallas as pl
from jax.experimental.pallas import tpu as pltpu

NUM_NODES = 10000
H_DIM = 128
OUT_DIM = 16
NUM_RELS = 16
E = 320000

_NB = 1000


def _matmul_body(h_ref, w_ref, out_ref):
    h = jnp.maximum(h_ref[...], 0.0)
    out_ref[0] = jnp.dot(h, w_ref[0], preferred_element_type=jnp.float32)


def _relu_matmul(h, w):
    n, d = h.shape
    r, _, f = w.shape
    grid = (n // _NB, r)
    return pl.pallas_call(
        _matmul_body,
        grid=grid,
        in_specs=[
            pl.BlockSpec((_NB, d), lambda i, j: (i, 0)),
            pl.BlockSpec((1, d, f), lambda i, j: (j, 0, 0)),
        ],
        out_specs=pl.BlockSpec((1, _NB, f), lambda i, j: (j, i, 0)),
        out_shape=jax.ShapeDtypeStruct((r, n, f), jnp.float32),
    )(h, w)


def _softmax_body(x_ref, out_ref):
    x = x_ref[...]
    m = jnp.max(x, axis=-1, keepdims=True)
    e = jnp.exp(x - m)
    out_ref[...] = e / jnp.sum(e, axis=-1, keepdims=True)


def _softmax(x):
    n, f = x.shape
    return pl.pallas_call(
        _softmax_body,
        grid=(n // _NB,),
        in_specs=[pl.BlockSpec((_NB, f), lambda i: (i, 0))],
        out_specs=pl.BlockSpec((_NB, f), lambda i: (i, 0)),
        out_shape=jax.ShapeDtypeStruct((n, f), jnp.float32),
    )(x)


def _gather_scale_segsum(table, gidx, dst, norm_e):
    msg = jnp.take(table, gidx, axis=0) * norm_e[:, None]
    return jax.ops.segment_sum(msg, dst, num_segments=NUM_NODES)


def kernel(edge_index, rel_type, norm, W0, W1, W2):
    src = edge_index[0]
    dst = edge_index[1]
    norm_e = norm[:, 0]
    gidx = rel_type * NUM_NODES + src

    h0 = _gather_scale_segsum(W0.reshape(-1, H_DIM), gidx, dst, norm_e)
    xw1 = _relu_matmul(h0, W1)
    h1 = _gather_scale_segsum(xw1.reshape(-1, H_DIM), gidx, dst, norm_e)
    xw2 = _relu_matmul(h1, W2)
    h2 = _gather_scale_segsum(xw2.reshape(-1, OUT_DIM), gidx, dst, norm_e)
    return _softmax(h2)


# speedup vs baseline: 6.1738x; 3.4080x over previous
import functools

import jax
import jax.numpy as jnp
from jax import lax
from jax.experimental import pallas as pl
from jax.experimental.pallas import tpu as pltpu
from jax.experimental.pallas import tpu_sc as plsc

NUM_NODES = 10000
H_DIM = 128
OUT_DIM = 16
NUM_RELS = 16
E = 320000

_NB = 1000


def _matmul_body(h_ref, w_ref, out_ref):
    h = jnp.maximum(h_ref[0] + h_ref[1], 0.0)
    out_ref[0] = jnp.dot(h, w_ref[0], preferred_element_type=jnp.float32)


def _relu_matmul(p, w):
    d = p.shape[-1]
    r, _, f = w.shape
    n = NUM_NODES
    grid = (n // _NB, r)
    return pl.pallas_call(
        _matmul_body,
        grid=grid,
        in_specs=[
            pl.BlockSpec((2, _NB, d), lambda i, j: (0, i, 0)),
            pl.BlockSpec((1, d, f), lambda i, j: (j, 0, 0)),
        ],
        out_specs=pl.BlockSpec((1, _NB, f), lambda i, j: (j, i, 0)),
        out_shape=jax.ShapeDtypeStruct((r, n, f), jnp.float32),
    )(p, w)


def _softmax_body(x_ref, out_ref):
    x = (x_ref[0] + x_ref[1])[:, :OUT_DIM]
    m = jnp.max(x, axis=-1, keepdims=True)
    e = jnp.exp(x - m)
    out_ref[...] = e / jnp.sum(e, axis=-1, keepdims=True)


def _softmax(p):
    f = p.shape[-1]
    n = NUM_NODES
    return pl.pallas_call(
        _softmax_body,
        grid=(n // _NB,),
        in_specs=[pl.BlockSpec((2, _NB, f), lambda i: (0, i, 0))],
        out_specs=pl.BlockSpec((_NB, OUT_DIM), lambda i: (i, 0)),
        out_shape=jax.ShapeDtypeStruct((n, OUT_DIM), jnp.float32),
    )(p)


_SC_B = 128
_NC, _NS = 2, 16
_NW = _NC * _NS
_NBLK = E // _SC_B
_NPAD = 10240
_RPS = _NPAD // _NS
_ZR = 128


@functools.partial(jax.jit, static_argnames=("f",))
def _sc_segsum(table, src, rel, dst, norm_e, f):
    fc = f // 16
    base_blocks = _NBLK // _NW
    extra = _NBLK - base_blocks * _NW
    mesh = plsc.VectorSubcoreMesh(core_axis_name="c", subcore_axis_name="s")

    @functools.partial(
        pl.kernel,
        mesh=mesh,
        out_type=jax.ShapeDtypeStruct((_NC, _NPAD, f), jnp.float32),
        scratch_types=[
            pltpu.VMEM((_SC_B,), jnp.int32),
            pltpu.VMEM((_SC_B,), jnp.int32),
            pltpu.VMEM((_SC_B,), jnp.int32),
            pltpu.VMEM((_SC_B,), jnp.float32),
            pltpu.VMEM((_SC_B,), jnp.int32),
            pltpu.VMEM((_SC_B, f), jnp.float32),
            pltpu.VMEM((_ZR, f), jnp.float32),
            pltpu.VMEM_SHARED((_NPAD, f), jnp.float32),
        ],
    )
    def k(table_h, src_h, rel_h, dst_h, norm_h, out_h,
          src_v, rel_v, dst_v, norm_v, gidx_v, rows_v, zbuf, acc):
        c = lax.axis_index("c")
        s = lax.axis_index("s")
        w = s * _NC + c

        zero16 = jnp.zeros((16,), jnp.float32)

        def zrow(i, carry):
            for j in range(fc):
                zbuf[i, pl.ds(16 * j, 16)] = zero16
            return carry

        lax.fori_loop(0, _ZR, zrow, 0)
        for kk in range(_RPS // _ZR):
            pltpu.sync_copy(zbuf, acc.at[pl.ds(s * _RPS + kk * _ZR, _ZR)])
        plsc.subcore_barrier()

        nblk = base_blocks + jnp.where(w < extra, 1, 0)

        def body(b, carry):
            base = (w + _NW * b) * _SC_B
            pltpu.sync_copy(src_h.at[pl.ds(base, _SC_B)], src_v)
            pltpu.sync_copy(rel_h.at[pl.ds(base, _SC_B)], rel_v)
            pltpu.sync_copy(dst_h.at[pl.ds(base, _SC_B)], dst_v)
            pltpu.sync_copy(norm_h.at[pl.ds(base, _SC_B)], norm_v)
            for j in range(_SC_B // 16):
                sl = pl.ds(16 * j, 16)
                gidx_v[sl] = rel_v[sl] * NUM_NODES + src_v[sl]
            pltpu.sync_copy(table_h.at[gidx_v], rows_v)

            def scale(g, carry2):
                nvec = norm_v[pl.ds(16 * g, 16)]
                for l in range(16):
                    nsp = jnp.take(nvec, jnp.full((16,), l, jnp.int32))
                    e = 16 * g + l
                    for j in range(fc):
                        sl = pl.ds(16 * j, 16)
                        rows_v[e, sl] = rows_v[e, sl] * nsp
                return carry2

            lax.fori_loop(0, _SC_B // 16, scale, 0)
            pltpu.sync_copy(rows_v, acc.at[dst_v], add=True)
            return carry

        lax.fori_loop(0, nblk, body, 0)
        plsc.subcore_barrier()
        pltpu.sync_copy(acc.at[pl.ds(s * _RPS, _RPS)],
                        out_h.at[c, pl.ds(s * _RPS, _RPS)])

    return k(table, src, rel, dst, norm_e)


def kernel(edge_index, rel_type, norm, W0, W1, W2):
    src = edge_index[0]
    dst = edge_index[1]
    norm_e = norm[:, 0]

    W2p = jnp.concatenate(
        [W2, jnp.zeros((NUM_RELS, H_DIM, H_DIM - OUT_DIM), jnp.float32)], axis=-1)

    h0 = _sc_segsum(W0.reshape(-1, H_DIM), src, rel_type, dst, norm_e, f=H_DIM)
    xw1 = _relu_matmul(h0, W1)
    h1 = _sc_segsum(xw1.reshape(-1, H_DIM), src, rel_type, dst, norm_e, f=H_DIM)
    xw2 = _relu_matmul(h1, W2p)
    h2 = _sc_segsum(xw2.reshape(-1, H_DIM), src, rel_type, dst, norm_e, f=H_DIM)
    return _softmax(h2)
